# baseline (speedup 1.0000x reference)
import functools

import jax
import jax.numpy as jnp
from jax import lax
from jax.experimental import pallas as pl
from jax.experimental.pallas import tpu as pltpu
from jax.experimental.pallas import tpu_sc as plsc

VOCAB = 100000
EMB = 128
B = 1024
S = 512
LANES = 16
NC = 2
NS = 16
NW = NC * NS
S_PER_W = S // NW
CHUNK = 128
NCHUNK = B // CHUNK
PAIR = 2
NROUND = S_PER_W * NCHUNK // PAIR


def _emb_body(idxT_hbm, table_hbm, pos_hbm, out_hbm,
              idx_all, pos_all, raw_v, rows_v, oidx_v, gsem, ssem):
    wid = lax.axis_index("s") * NC + lax.axis_index("c")
    s0 = wid * S_PER_W
    pltpu.sync_copy(idxT_hbm.at[pl.ds(s0, S_PER_W)], idx_all)
    pltpu.sync_copy(pos_hbm.at[pl.ds(s0, S_PER_W)], pos_all)

    def gather_of(r, half, b):
        t = PAIR * r + b
        si = t >> 3
        c = t & (NCHUNK - 1)
        slot = half * PAIR + b
        return pltpu.make_async_copy(
            table_hbm.at[idx_all.at[si, pl.ds(c * CHUNK, CHUNK)]],
            raw_v.at[slot], gsem.at[slot])

    def scatter_of(half, b):
        slot = half * PAIR + b
        return pltpu.make_async_copy(
            rows_v.at[slot], out_hbm.at[oidx_v.at[slot]], ssem.at[slot])

    def launch_half(r, half):
        for b in range(PAIR):
            gather_of(r, half, b).start()

    def wait_scatters(half):
        for b in range(PAIR):
            scatter_of(half, b).wait()

    def process_half(r, half):
        for b in range(PAIR):
            slot = half * PAIR + b
            t = PAIR * r + b
            si = t >> 3
            c = t & (NCHUNK - 1)
            s = s0 + si
            lane = lax.iota(jnp.int32, LANES)
            for k in range(CHUNK // LANES):
                oidx_v[slot, pl.ds(k * LANES, LANES)] = (
                    lane * S + (c * CHUNK + k * LANES) * S + s)
            pvecs = [pos_all[si, pl.ds(j * LANES, LANES)]
                     for j in range(EMB // LANES)]
            gather_of(r, half, b).wait()

            @plsc.parallel_loop(0, CHUNK, unroll=4)
            def _(tt):
                for k in range(EMB // (2 * LANES)):
                    w = raw_v[slot, tt, pl.ds(LANES * k, LANES)]
                    even = lax.bitcast_convert_type(w << 16, jnp.float32)
                    odd = lax.bitcast_convert_type((w >> 16) << 16, jnp.float32)
                    rows_v[slot, tt, pl.ds(2 * LANES * k, LANES)] = (
                        even + pvecs[2 * k])
                    rows_v[slot, tt, pl.ds(2 * LANES * k + LANES, LANES)] = (
                        odd + pvecs[2 * k + 1])

            scatter_of(half, b).start()

    def body(gg, carry):
        r0 = 2 * gg
        r1 = 2 * gg + 1

        @pl.when(gg >= 1)
        def _():
            wait_scatters(0)
            launch_half(r0, 0)
            process_half(r0 - 1, 1)
            wait_scatters(1)

        @pl.when(gg == 0)
        def _():
            launch_half(r0, 0)

        launch_half(r1, 1)
        process_half(r0, 0)
        return carry

    lax.fori_loop(0, NROUND // 2, body, 0)
    process_half(NROUND - 1, 1)
    wait_scatters(0)
    wait_scatters(1)


_emb = functools.partial(
    pl.kernel,
    out_type=jax.ShapeDtypeStruct((B * S, EMB), jnp.float32),
    mesh=plsc.VectorSubcoreMesh(core_axis_name="c", subcore_axis_name="s"),
    compiler_params=pltpu.CompilerParams(use_tc_tiling_on_sc=False),
    scratch_types=[
        pltpu.VMEM((S_PER_W, B), jnp.int32),
        pltpu.VMEM((S_PER_W, EMB), jnp.float32),
        pltpu.VMEM((2 * PAIR, CHUNK, EMB // 2), jnp.uint32),
        pltpu.VMEM((2 * PAIR, CHUNK, EMB), jnp.float32),
        pltpu.VMEM((2 * PAIR, CHUNK), jnp.int32),
        pltpu.SemaphoreType.DMA((2 * PAIR,)),
        pltpu.SemaphoreType.DMA((2 * PAIR,)),
    ],
)(_emb_body)


def kernel(inputs, token_table, position_embedding):
    idxT = jnp.transpose(inputs.astype(jnp.int32))
    packed = (token_table.reshape(VOCAB, EMB // 32, 2, LANES)
              .transpose(0, 1, 3, 2)
              .reshape(VOCAB, EMB // 2, 2)
              .astype(jnp.bfloat16))
    packed = lax.bitcast_convert_type(packed, jnp.uint32)
    out = _emb(idxT, packed, position_embedding[:S])
    return out.reshape(B, S, EMB)


# speedup vs baseline: 1.5870x; 1.5870x over previous
import functools

import jax
import jax.numpy as jnp
from jax import lax
from jax.experimental import pallas as pl
from jax.experimental.pallas import tpu as pltpu
from jax.experimental.pallas import tpu_sc as plsc

VOCAB = 100000
EMB = 128
B = 1024
S = 512
LANES = 16
NC = 2
NS = 16
NW = NC * NS
S_PER_W = S // NW
CHUNK = 128
NCHUNK = B // CHUNK
PAIR = 2
NROUND = S_PER_W * NCHUNK // PAIR


def _emb_body(idxT_hbm, table_hbm, pos_hbm, out_hbm,
              idx_all, pos_all, rows_v, oidx_v, gsem, ssem):
    wid = lax.axis_index("s") * NC + lax.axis_index("c")
    s0 = wid * S_PER_W
    pltpu.sync_copy(idxT_hbm.at[pl.ds(s0, S_PER_W)], idx_all)
    pltpu.sync_copy(pos_hbm.at[pl.ds(s0, S_PER_W)], pos_all)

    def gather_of(r, half, b):
        t = PAIR * r + b
        si = t >> 3
        c = t & (NCHUNK - 1)
        slot = half * PAIR + b
        return pltpu.make_async_copy(
            table_hbm.at[idx_all.at[si, pl.ds(c * CHUNK, CHUNK)]],
            rows_v.at[slot], gsem.at[slot])

    def scatter_of(half, b):
        slot = half * PAIR + b
        return pltpu.make_async_copy(
            rows_v.at[slot], out_hbm.at[oidx_v.at[slot]], ssem.at[slot])

    def launch_half(r, half):
        for b in range(PAIR):
            gather_of(r, half, b).start()

    def wait_scatters(half):
        for b in range(PAIR):
            scatter_of(half, b).wait()

    def process_half(r, half):
        for b in range(PAIR):
            slot = half * PAIR + b
            t = PAIR * r + b
            si = t >> 3
            c = t & (NCHUNK - 1)
            s = s0 + si
            lane = lax.iota(jnp.int32, LANES)
            for k in range(CHUNK // LANES):
                oidx_v[slot, pl.ds(k * LANES, LANES)] = (
                    lane * S + (c * CHUNK + k * LANES) * S + s)
            pvecs = [pos_all[si, pl.ds(j * LANES, LANES)]
                     for j in range(EMB // LANES)]
            gather_of(r, half, b).wait()

            @plsc.parallel_loop(0, CHUNK, unroll=4)
            def _(tt):
                for j in range(EMB // LANES):
                    plsc.addupdate(
                        rows_v.at[slot, tt, pl.ds(j * LANES, LANES)], pvecs[j])

            scatter_of(half, b).start()

    def body(gg, carry):
        r0 = 2 * gg
        r1 = 2 * gg + 1

        @pl.when(gg >= 1)
        def _():
            wait_scatters(0)
            launch_half(r0, 0)
            process_half(r0 - 1, 1)
            wait_scatters(1)

        @pl.when(gg == 0)
        def _():
            launch_half(r0, 0)

        launch_half(r1, 1)
        process_half(r0, 0)
        return carry

    lax.fori_loop(0, NROUND // 2, body, 0)
    process_half(NROUND - 1, 1)
    wait_scatters(0)
    wait_scatters(1)


_emb = functools.partial(
    pl.kernel,
    out_type=jax.ShapeDtypeStruct((B * S, EMB), jnp.float32),
    mesh=plsc.VectorSubcoreMesh(core_axis_name="c", subcore_axis_name="s"),
    scratch_types=[
        pltpu.VMEM((S_PER_W, B), jnp.int32),
        pltpu.VMEM((S_PER_W, EMB), jnp.float32),
        pltpu.VMEM((2 * PAIR, CHUNK, EMB), jnp.float32),
        pltpu.VMEM((2 * PAIR, CHUNK), jnp.int32),
        pltpu.SemaphoreType.DMA((2 * PAIR,)),
        pltpu.SemaphoreType.DMA((2 * PAIR,)),
    ],
)(_emb_body)


def kernel(inputs, token_table, position_embedding):
    idxT = jnp.transpose(inputs.astype(jnp.int32))
    out = _emb(idxT, token_table, position_embedding[:S])
    return out.reshape(B, S, EMB)
